# Initial kernel scaffold; baseline (speedup 1.0000x reference)
import jax
import jax.numpy as jnp
from jax.experimental import pallas as pl


def kernel(deltas, logits, anchors):
    raise NotImplementedError("write your pallas kernel here")


# speedup vs baseline: 69.3575x; 69.3575x over previous
import functools

import jax
import jax.numpy as jnp
from jax import lax
from jax.experimental import pallas as pl
from jax.experimental.pallas import tpu as pltpu

_CONF_TH = 0.05
_NMS_TH = 0.5
_L = 128
_R = 40
_NP = _R * _L
_B = 4
_NG = _B * _NP
_NW = 32
_PER = _NG // _NW
_NCH = _PER // _L
_NCHP = 8
_JC = 1024
_ND = _NP // _JC


def _col1(row):
    return jnp.transpose(jnp.broadcast_to(row, (_L, _L)), (1, 0))[:, 0:1]


def _cols(v, width=_L):
    cols = [_col1(v[c:c + 1, :]) for c in range(v.shape[0])]
    per = width // _L
    return [jnp.concatenate(cols[d * per:(d + 1) * per], axis=0)
            for d in range(len(cols) // per)]


def _stage_a(lg_ref, dl_ref, an_ref,
             x1o, y1o, x2o, y2o, sco, vao, rko, odo,
             pr_r, pc, ic, rc, sa, sb):
    ml = jnp.max(lg_ref[0], axis=0)
    score = jax.nn.sigmoid(ml)
    ax1, ay1, ax2, ay2 = an_ref[0], an_ref[1], an_ref[2], an_ref[3]
    aw = ax2 - ax1
    ah = ay2 - ay1
    acx = ax1 + 0.5 * aw
    acy = ay1 + 0.5 * ah
    dx = dl_ref[0, 0]
    dy = dl_ref[0, 1]
    dw = jnp.minimum(dl_ref[0, 2], 4.0)
    dh = jnp.minimum(dl_ref[0, 3], 4.0)
    pcx = dx * aw + acx
    pcy = dy * ah + acy
    pw = jnp.exp(dw) * aw
    ph = jnp.exp(dh) * ah
    x1 = jnp.clip(pcx - 0.5 * pw, 0.0, 1.0)
    y1 = jnp.clip(pcy - 0.5 * ph, 0.0, 1.0)
    x2 = jnp.clip(pcx + 0.5 * pw, 0.0, 1.0)
    y2 = jnp.clip(pcy + 0.5 * ph, 0.0, 1.0)
    w = x2 - x1
    h = y2 - y1
    valid = ((score > _CONF_TH) & (w > 0.01) & (h > 0.01)
             & (w < 0.99) & (h < 0.99))
    pri = jnp.where(valid, score, -1.0)

    x1o[0] = x1
    y1o[0] = y1
    x2o[0] = x2
    y2o[0] = y2
    sco[0] = score
    vao[0] = valid.astype(jnp.float32)

    pr_r[...] = pri.reshape(_R, 1, _L)
    pchunks = _cols(pri, _JC)
    for d in range(_ND):
        pc[d] = pchunks[d]

    base = pl.program_id(0) * _NP

    sa[...] = jnp.zeros((_R, 1, _L), jnp.float32)

    def rank_tile(t, carry):
        c = t // _ND
        d = t % _ND
        pi = pr_r[c]
        ii = lax.broadcasted_iota(jnp.int32, (1, _L), 1) + c * _L
        pj = pc[d]
        jj = lax.broadcasted_iota(jnp.int32, (_JC, 1), 0) + d * _JC
        prio = (pj > pi) | ((pj == pi) & (jj < ii))
        cnt = jnp.sum(prio.astype(jnp.float32), axis=0, keepdims=True)
        sa[c] = sa[c] + cnt
        return carry

    lax.fori_loop(0, _R * _ND, rank_tile, 0)
    rank_f = sa[...].reshape(_R, _L)
    rko[0] = rank_f.astype(jnp.int32) + base

    rchunks = _cols(rank_f, _JC)
    for d in range(_ND):
        rc[d] = rchunks[d]
        ic[d] = (lax.broadcasted_iota(jnp.int32, (_JC, 1), 0)
                 .astype(jnp.float32) + jnp.float32(d * _JC))
    sb[...] = jnp.zeros((_R, 1, _L), jnp.float32)

    def order_tile(t, carry):
        c = t // _ND
        d = t % _ND
        ss = (lax.broadcasted_iota(jnp.int32, (1, _L), 1)
              .astype(jnp.float32) + jnp.float32(c * _L))
        eq = (rc[d] == ss).astype(jnp.float32)
        sb[c] = sb[c] + jnp.sum(eq * ic[d], axis=0, keepdims=True)
        return carry

    lax.fori_loop(0, _R * _ND, order_tile, 0)
    odo[0] = sb[...].reshape(_R, _L).astype(jnp.int32) + base


def _stage_c(x1_ref, y1_ref, x2_ref, y2_ref, va_ref, keep_o,
             xr, yr, Xr, Yr, ar, vr, xc, yc, Xc, Yc, ac, kc):
    x1, y1 = x1_ref[0], y1_ref[0]
    x2, y2 = x2_ref[0], y2_ref[0]
    va = va_ref[0]
    area = (x2 - x1) * (y2 - y1)
    for ref, v in ((xr, x1), (yr, y1), (Xr, x2), (Yr, y2),
                   (ar, area), (vr, va)):
        ref[...] = v.reshape(_R, 1, _L)
    for ref, v in ((xc, x1), (yc, y1), (Xc, x2), (Yc, y2), (ac, area)):
        chunks = _cols(v, _L)
        for d in range(_R):
            ref[d] = chunks[d]

    def block(b, carry):
        xi, yi, Xi, Yi = xr[b], yr[b], Xr[b], Yr[b]
        ai, vi = ar[b], vr[b]

        def cross(d, supp):
            ltx = jnp.maximum(xc[d], xi)
            lty = jnp.maximum(yc[d], yi)
            rbx = jnp.minimum(Xc[d], Xi)
            rby = jnp.minimum(Yc[d], Yi)
            wx = jnp.maximum(rbx - ltx, 0.0)
            wy = jnp.maximum(rby - lty, 0.0)
            inter = wx * wy
            iou = inter / (ac[d] + ai - inter + 1e-9)
            m = (iou > _NMS_TH) & (kc[d] > 0.5)
            return jnp.maximum(
                supp, jnp.any(m, axis=0, keepdims=True).astype(jnp.float32))

        supp = lax.fori_loop(0, b, cross, jnp.zeros((1, _L), jnp.float32))
        basek = jnp.where((vi > 0.5) & (supp < 0.5), 1.0, 0.0)

        xjb, yjb = _col1(xi), _col1(yi)
        Xjb, Yjb = _col1(Xi), _col1(Yi)
        ajb = _col1(ai)
        ltx = jnp.maximum(xjb, xi)
        lty = jnp.maximum(yjb, yi)
        rbx = jnp.minimum(Xjb, Xi)
        rby = jnp.minimum(Yjb, Yi)
        wx = jnp.maximum(rbx - ltx, 0.0)
        wy = jnp.maximum(rby - lty, 0.0)
        inter = wx * wy
        iou = inter / (ajb + ai - inter + 1e-9)
        jl = lax.broadcasted_iota(jnp.int32, (_L, _L), 0)
        il = lax.broadcasted_iota(jnp.int32, (_L, _L), 1)
        adj = (iou > _NMS_TH) & (jl < il)

        def fcond(s):
            return s[1]

        def fbody(s):
            k, _ = s
            kcol = _col1(k) > 0.5
            sup = jnp.any(adj & kcol, axis=0, keepdims=True)
            nk = jnp.where(sup, 0.0, basek)
            return nk, jnp.any(nk != k)

        kblk, _ = lax.while_loop(fcond, fbody, (basek, jnp.bool_(True)))

        keep_o[0, b] = kblk
        kc[b] = _col1(kblk)
        return carry


    lax.fori_loop(0, _R, block, 0)


def _make_sc_gather(n_tables):
    from jax.experimental.pallas import tpu_sc as plsc
    mesh = plsc.VectorSubcoreMesh(core_axis_name="c", subcore_axis_name="s")

    @functools.partial(
        pl.kernel, mesh=mesh,
        out_type=[jax.ShapeDtypeStruct((_NW * _NCHP, _L), jnp.float32)
                  for _ in range(n_tables)],
        scratch_types=[
            pltpu.VMEM((_NCHP, _L), jnp.int32),
            pltpu.VMEM((_NCHP, _L), jnp.float32),
            pltpu.SemaphoreType.DMA,
        ],
    )
    def k(idx_hbm, *rest):
        tables = rest[:n_tables]
        outs = rest[n_tables:2 * n_tables]
        idx_v, val_v, sem = rest[2 * n_tables:]
        wid = lax.axis_index("s") * 2 + lax.axis_index("c")
        pltpu.sync_copy(idx_hbm.at[pl.ds(wid * _NCHP, _NCHP)], idx_v)
        for a in range(n_tables):
            for j in range(_NCH):
                pltpu.async_copy(tables[a].at[idx_v.at[j]], val_v.at[j], sem)
            for j in range(_NCH):
                pltpu.make_async_copy(tables[a].at[idx_v.at[0]],
                                      val_v.at[0], sem).wait()
            pltpu.sync_copy(val_v, outs[a].at[pl.ds(wid * _NCHP, _NCHP)])
    return k


def _make_sc_gather_mul(n_tables):
    from jax.experimental.pallas import tpu_sc as plsc
    mesh = plsc.VectorSubcoreMesh(core_axis_name="c", subcore_axis_name="s")

    @functools.partial(
        pl.kernel, mesh=mesh,
        out_type=[jax.ShapeDtypeStruct((_NW * _NCHP, _L), jnp.float32)
                  for _ in range(n_tables)],
        scratch_types=[
            pltpu.VMEM((_NCHP, _L), jnp.int32),
            pltpu.VMEM((_NCHP, _L), jnp.float32),
            pltpu.VMEM((_NCHP, _L), jnp.float32),
            pltpu.SemaphoreType.DMA,
        ],
    )
    def k(idx_hbm, keep_hbm, *rest):
        lins = rest[:n_tables]
        outs = rest[n_tables:2 * n_tables]
        idx_v, kv, av, sem = rest[2 * n_tables:]
        wid = lax.axis_index("s") * 2 + lax.axis_index("c")
        pltpu.sync_copy(idx_hbm.at[pl.ds(wid * _NCHP, _NCHP)], idx_v)
        for j in range(_NCH):
            pltpu.async_copy(keep_hbm.at[idx_v.at[j]], kv.at[j], sem)
        for j in range(_NCH):
            pltpu.make_async_copy(keep_hbm.at[idx_v.at[0]],
                                  kv.at[0], sem).wait()
        for a in range(n_tables):
            pltpu.sync_copy(lins[a].at[pl.ds(wid * _NCHP, _NCHP)], av)
            for r in range(_NCH):
                for l in range(_L // 16):
                    av[r, pl.ds(l * 16, 16)] = (av[r, pl.ds(l * 16, 16)]
                                                * kv[r, pl.ds(l * 16, 16)])
            pltpu.sync_copy(av, outs[a].at[pl.ds(wid * _NCHP, _NCHP)])
    return k


def _pad_lanes(flat):
    v = flat.reshape(_NW, _NCH, _L)
    v = jnp.pad(v, ((0, 0), (0, _NCHP - _NCH), (0, 0)))
    return v.reshape(_NW * _NCHP, _L)


def _unpad_lanes(padded):
    return padded.reshape(_NW, _NCHP, _L)[:, :_NCH].reshape(_NG)


@jax.jit
def kernel(deltas, logits, anchors):
    B, N, C = logits.shape
    pad = _NP - N
    logits_p = jnp.pad(logits, ((0, 0), (0, pad), (0, 0)),
                       constant_values=-30.0)
    deltas_p = jnp.pad(deltas, ((0, 0), (0, pad), (0, 0)))
    anchors_p = jnp.pad(anchors, ((0, pad), (0, 0)))
    logits_t = logits_p.transpose(0, 2, 1).reshape(B, C, _R, _L)
    deltas_t = deltas_p.transpose(0, 2, 1).reshape(B, 4, _R, _L)
    anchors_t = anchors_p.T.reshape(4, _R, _L)

    row = pltpu.VMEM((_R, 1, _L), jnp.float32)
    colJ = pltpu.VMEM((_ND, _JC, 1), jnp.float32)
    colL = pltpu.VMEM((_R, _L, 1), jnp.float32)
    bs = pl.BlockSpec((1, _R, _L), lambda b: (b, 0, 0))

    outs = pl.pallas_call(
        _stage_a,
        grid=(B,),
        in_specs=[
            pl.BlockSpec((1, C, _R, _L), lambda b: (b, 0, 0, 0)),
            pl.BlockSpec((1, 4, _R, _L), lambda b: (b, 0, 0, 0)),
            pl.BlockSpec((4, _R, _L), lambda b: (0, 0, 0)),
        ],
        out_specs=[bs] * 8,
        out_shape=([jax.ShapeDtypeStruct((B, _R, _L), jnp.float32)] * 6
                   + [jax.ShapeDtypeStruct((B, _R, _L), jnp.int32)] * 2),
        scratch_shapes=[row, colJ, colJ, colJ, row, row],
    )(logits_t, deltas_t, anchors_t)
    x1, y1, x2, y2, score, validf, rank, order = outs

    gather5 = _make_sc_gather(5)
    idx_o = _pad_lanes(order.reshape(_NG))
    tabs = [v.reshape(_NG) for v in (x1, y1, x2, y2, validf)]
    sx1, sy1, sx2, sy2, sva = gather5(idx_o, *tabs)
    sx1, sy1, sx2, sy2, sva = (
        _unpad_lanes(v).reshape(B, _R, _L) for v in (sx1, sy1, sx2, sy2, sva))

    keep_s = pl.pallas_call(
        _stage_c,
        grid=(B,),
        in_specs=[bs] * 5,
        out_specs=pl.BlockSpec((1, _R, 1, _L), lambda b: (b, 0, 0, 0)),
        out_shape=jax.ShapeDtypeStruct((B, _R, 1, _L), jnp.float32),
        scratch_shapes=[row, row, row, row, row, row,
                        colL, colL, colL, colL, colL, colL],
    )(sx1, sy1, sx2, sy2, sva)

    gm = _make_sc_gather_mul(5)
    idx_r = _pad_lanes(rank.reshape(_NG))
    lins = [_pad_lanes(v.reshape(_NG)) for v in (x1, y1, x2, y2, score)]
    mx1, my1, mx2, my2, msc = gm(idx_r, keep_s.reshape(_NG), *lins)
    parts = [_unpad_lanes(v).reshape(B, _NP)[:, :N]
             for v in (mx1, my1, mx2, my2, msc)]
    return jnp.stack(parts, axis=-1)


# speedup vs baseline: 72.7802x; 1.0493x over previous
import functools

import jax
import jax.numpy as jnp
from jax import lax
from jax.experimental import pallas as pl
from jax.experimental.pallas import tpu as pltpu

_CONF_TH = 0.05
_NMS_TH = 0.5
_L = 128
_R = 40
_NP = _R * _L
_B = 4
_NG = _B * _NP
_NW = 32
_PER = _NG // _NW
_NCH = _PER // _L
_NCHP = 8


def _col1(row):
    return jnp.transpose(jnp.broadcast_to(row, (_L, _L)), (1, 0))[:, 0:1]


def _stage_a(lg_ref, dl_ref, an_ref,
             x1o, y1o, x2o, y2o, sco, vao, rko, odo,
             pr_r, pc, rc, sa, sb):
    ml = jnp.max(lg_ref[0], axis=0)
    score = jax.nn.sigmoid(ml)
    ax1, ay1, ax2, ay2 = an_ref[0], an_ref[1], an_ref[2], an_ref[3]
    aw = ax2 - ax1
    ah = ay2 - ay1
    acx = ax1 + 0.5 * aw
    acy = ay1 + 0.5 * ah
    dx = dl_ref[0, 0]
    dy = dl_ref[0, 1]
    dw = jnp.minimum(dl_ref[0, 2], 4.0)
    dh = jnp.minimum(dl_ref[0, 3], 4.0)
    pcx = dx * aw + acx
    pcy = dy * ah + acy
    pw = jnp.exp(dw) * aw
    ph = jnp.exp(dh) * ah
    x1 = jnp.clip(pcx - 0.5 * pw, 0.0, 1.0)
    y1 = jnp.clip(pcy - 0.5 * ph, 0.0, 1.0)
    x2 = jnp.clip(pcx + 0.5 * pw, 0.0, 1.0)
    y2 = jnp.clip(pcy + 0.5 * ph, 0.0, 1.0)
    w = x2 - x1
    h = y2 - y1
    valid = ((score > _CONF_TH) & (w > 0.01) & (h > 0.01)
             & (w < 0.99) & (h < 0.99))
    pri = jnp.where(valid, score, -1.0)

    x1o[0] = x1
    y1o[0] = y1
    x2o[0] = x2
    y2o[0] = y2
    sco[0] = score
    vao[0] = valid.astype(jnp.float32)

    pr_r[...] = pri.reshape(_R, 1, _L)
    pc[...] = jnp.concatenate([_col1(pri[c:c + 1, :]) for c in range(_R)],
                              axis=0)

    base = pl.program_id(0) * _NP

    def rank_tile(c, carry):
        pi = pr_r[c]
        ii = lax.broadcasted_iota(jnp.int32, (1, _L), 1) + c * _L
        pj = pc[...]
        jj = lax.broadcasted_iota(jnp.int32, (_NP, 1), 0)
        prio = (pj > pi) | ((pj == pi) & (jj < ii))
        sa[c] = jnp.sum(prio.astype(jnp.float32), axis=0, keepdims=True)
        return carry

    lax.fori_loop(0, _R, rank_tile, 0)
    rank_f = sa[...].reshape(_R, _L)
    rko[0] = rank_f.astype(jnp.int32) + base

    rc[...] = jnp.concatenate([_col1(rank_f[c:c + 1, :]) for c in range(_R)],
                              axis=0)

    def order_tile(c, carry):
        ss = (lax.broadcasted_iota(jnp.int32, (1, _L), 1)
              .astype(jnp.float32) + jnp.float32(c * _L))
        idxc = lax.broadcasted_iota(jnp.int32, (_NP, 1), 0).astype(jnp.float32)
        eq = (rc[...] == ss).astype(jnp.float32)
        sb[c] = jnp.sum(eq * idxc, axis=0, keepdims=True)
        return carry

    lax.fori_loop(0, _R, order_tile, 0)
    odo[0] = sb[...].reshape(_R, _L).astype(jnp.int32) + base


def _stage_c(x1_ref, y1_ref, x2_ref, y2_ref, va_ref, keep_o,
             xr, yr, Xr, Yr, ar, vr, xc, yc, Xc, Yc, ac, kc):
    x1, y1 = x1_ref[0], y1_ref[0]
    x2, y2 = x2_ref[0], y2_ref[0]
    va = va_ref[0]
    area = (x2 - x1) * (y2 - y1)
    for ref, v in ((xr, x1), (yr, y1), (Xr, x2), (Yr, y2),
                   (ar, area), (vr, va)):
        ref[...] = v.reshape(_R, 1, _L)
    for ref, v in ((xc, x1), (yc, y1), (Xc, x2), (Yc, y2), (ac, area)):
        ref[...] = jnp.concatenate(
            [_col1(v[c:c + 1, :]) for c in range(_R)], axis=0)
    kc[...] = jnp.zeros((_NP, 1), jnp.float32)

    def block(b, carry):
        xi, yi, Xi, Yi = xr[b], yr[b], Xr[b], Yr[b]
        ai, vi = ar[b], vr[b]

        ltx = jnp.maximum(xc[...], xi)
        lty = jnp.maximum(yc[...], yi)
        rbx = jnp.minimum(Xc[...], Xi)
        rby = jnp.minimum(Yc[...], Yi)
        wx = jnp.maximum(rbx - ltx, 0.0)
        wy = jnp.maximum(rby - lty, 0.0)
        inter = wx * wy
        iou = inter / (ac[...] + ai - inter + 1e-9)
        m = (iou > _NMS_TH) & (kc[...] > 0.5)
        supp = jnp.any(m, axis=0, keepdims=True).astype(jnp.float32)
        basek = jnp.where((vi > 0.5) & (supp < 0.5), 1.0, 0.0)

        xjb, yjb = _col1(xi), _col1(yi)
        Xjb, Yjb = _col1(Xi), _col1(Yi)
        ajb = _col1(ai)
        ltx = jnp.maximum(xjb, xi)
        lty = jnp.maximum(yjb, yi)
        rbx = jnp.minimum(Xjb, Xi)
        rby = jnp.minimum(Yjb, Yi)
        wx = jnp.maximum(rbx - ltx, 0.0)
        wy = jnp.maximum(rby - lty, 0.0)
        inter = wx * wy
        iou = inter / (ajb + ai - inter + 1e-9)
        jl = lax.broadcasted_iota(jnp.int32, (_L, _L), 0)
        il = lax.broadcasted_iota(jnp.int32, (_L, _L), 1)
        adj = (iou > _NMS_TH) & (jl < il)

        def fcond(s):
            return s[1]

        def fbody(s):
            k, _ = s
            kcol = _col1(k) > 0.5
            sup = jnp.any(adj & kcol, axis=0, keepdims=True)
            nk = jnp.where(sup, 0.0, basek)
            return nk, jnp.any(nk != k)

        kblk, _ = lax.while_loop(fcond, fbody, (basek, jnp.bool_(True)))

        keep_o[0, b] = kblk
        kc[pl.ds(b * _L, _L), :] = _col1(kblk)
        return carry


    lax.fori_loop(0, _R, block, 0)


def _make_sc_gather(n_tables):
    from jax.experimental.pallas import tpu_sc as plsc
    mesh = plsc.VectorSubcoreMesh(core_axis_name="c", subcore_axis_name="s")

    @functools.partial(
        pl.kernel, mesh=mesh,
        out_type=[jax.ShapeDtypeStruct((_NW * _NCHP, _L), jnp.float32)
                  for _ in range(n_tables)],
        scratch_types=[
            pltpu.VMEM((_NCHP, _L), jnp.int32),
            pltpu.VMEM((_NCHP, _L), jnp.float32),
            pltpu.SemaphoreType.DMA,
        ],
    )
    def k(idx_hbm, *rest):
        tables = rest[:n_tables]
        outs = rest[n_tables:2 * n_tables]
        idx_v, val_v, sem = rest[2 * n_tables:]
        wid = lax.axis_index("s") * 2 + lax.axis_index("c")
        pltpu.sync_copy(idx_hbm.at[pl.ds(wid * _NCHP, _NCHP)], idx_v)
        for a in range(n_tables):
            for j in range(_NCH):
                pltpu.async_copy(tables[a].at[idx_v.at[j]], val_v.at[j], sem)
            for j in range(_NCH):
                pltpu.make_async_copy(tables[a].at[idx_v.at[0]],
                                      val_v.at[0], sem).wait()
            pltpu.sync_copy(val_v, outs[a].at[pl.ds(wid * _NCHP, _NCHP)])
    return k


def _make_sc_gather_mul(n_tables):
    from jax.experimental.pallas import tpu_sc as plsc
    mesh = plsc.VectorSubcoreMesh(core_axis_name="c", subcore_axis_name="s")

    @functools.partial(
        pl.kernel, mesh=mesh,
        out_type=[jax.ShapeDtypeStruct((_NW * _NCHP, _L), jnp.float32)
                  for _ in range(n_tables)],
        scratch_types=[
            pltpu.VMEM((_NCHP, _L), jnp.int32),
            pltpu.VMEM((_NCHP, _L), jnp.float32),
            pltpu.VMEM((_NCHP, _L), jnp.float32),
            pltpu.SemaphoreType.DMA,
        ],
    )
    def k(idx_hbm, keep_hbm, *rest):
        lins = rest[:n_tables]
        outs = rest[n_tables:2 * n_tables]
        idx_v, kv, av, sem = rest[2 * n_tables:]
        wid = lax.axis_index("s") * 2 + lax.axis_index("c")
        pltpu.sync_copy(idx_hbm.at[pl.ds(wid * _NCHP, _NCHP)], idx_v)
        for j in range(_NCH):
            pltpu.async_copy(keep_hbm.at[idx_v.at[j]], kv.at[j], sem)
        for j in range(_NCH):
            pltpu.make_async_copy(keep_hbm.at[idx_v.at[0]],
                                  kv.at[0], sem).wait()
        for a in range(n_tables):
            pltpu.sync_copy(lins[a].at[pl.ds(wid * _NCHP, _NCHP)], av)
            for r in range(_NCH):
                for l in range(_L // 16):
                    av[r, pl.ds(l * 16, 16)] = (av[r, pl.ds(l * 16, 16)]
                                                * kv[r, pl.ds(l * 16, 16)])
            pltpu.sync_copy(av, outs[a].at[pl.ds(wid * _NCHP, _NCHP)])
    return k


def _pad_lanes(flat):
    v = flat.reshape(_NW, _NCH, _L)
    v = jnp.pad(v, ((0, 0), (0, _NCHP - _NCH), (0, 0)))
    return v.reshape(_NW * _NCHP, _L)


def _unpad_lanes(padded):
    return padded.reshape(_NW, _NCHP, _L)[:, :_NCH].reshape(_NG)


@jax.jit
def kernel(deltas, logits, anchors):
    B, N, C = logits.shape
    pad = _NP - N
    logits_p = jnp.pad(logits, ((0, 0), (0, pad), (0, 0)),
                       constant_values=-30.0)
    deltas_p = jnp.pad(deltas, ((0, 0), (0, pad), (0, 0)))
    anchors_p = jnp.pad(anchors, ((0, pad), (0, 0)))
    logits_t = logits_p.transpose(0, 2, 1).reshape(B, C, _R, _L)
    deltas_t = deltas_p.transpose(0, 2, 1).reshape(B, 4, _R, _L)
    anchors_t = anchors_p.T.reshape(4, _R, _L)

    row = pltpu.VMEM((_R, 1, _L), jnp.float32)
    colF = pltpu.VMEM((_NP, 1), jnp.float32)
    bs = pl.BlockSpec((1, _R, _L), lambda b: (b, 0, 0))

    outs = pl.pallas_call(
        _stage_a,
        grid=(B,),
        in_specs=[
            pl.BlockSpec((1, C, _R, _L), lambda b: (b, 0, 0, 0)),
            pl.BlockSpec((1, 4, _R, _L), lambda b: (b, 0, 0, 0)),
            pl.BlockSpec((4, _R, _L), lambda b: (0, 0, 0)),
        ],
        out_specs=[bs] * 8,
        out_shape=([jax.ShapeDtypeStruct((B, _R, _L), jnp.float32)] * 6
                   + [jax.ShapeDtypeStruct((B, _R, _L), jnp.int32)] * 2),
        scratch_shapes=[row, colF, colF, row, row],
    )(logits_t, deltas_t, anchors_t)
    x1, y1, x2, y2, score, validf, rank, order = outs

    gather5 = _make_sc_gather(5)
    idx_o = _pad_lanes(order.reshape(_NG))
    tabs = [v.reshape(_NG) for v in (x1, y1, x2, y2, validf)]
    sx1, sy1, sx2, sy2, sva = gather5(idx_o, *tabs)
    sx1, sy1, sx2, sy2, sva = (
        _unpad_lanes(v).reshape(B, _R, _L) for v in (sx1, sy1, sx2, sy2, sva))

    keep_s = pl.pallas_call(
        _stage_c,
        grid=(B,),
        in_specs=[bs] * 5,
        out_specs=pl.BlockSpec((1, _R, 1, _L), lambda b: (b, 0, 0, 0)),
        out_shape=jax.ShapeDtypeStruct((B, _R, 1, _L), jnp.float32),
        scratch_shapes=[row, row, row, row, row, row,
                        colF, colF, colF, colF, colF, colF],
    )(sx1, sy1, sx2, sy2, sva)

    gm = _make_sc_gather_mul(5)
    idx_r = _pad_lanes(rank.reshape(_NG))
    lins = [_pad_lanes(v.reshape(_NG)) for v in (x1, y1, x2, y2, score)]
    mx1, my1, mx2, my2, msc = gm(idx_r, keep_s.reshape(_NG), *lins)
    parts = [_unpad_lanes(v).reshape(B, _NP)[:, :N]
             for v in (mx1, my1, mx2, my2, msc)]
    return jnp.stack(parts, axis=-1)
